# baseline (speedup 1.0000x reference)
import functools

import jax
import jax.numpy as jnp
import numpy as np
from jax import lax
from jax.experimental import pallas as pl
from jax.experimental.pallas import tpu as pltpu
from jax.experimental.pallas import tpu_sc as plsc

_C = 256
_NB = 1000
_NBP = 1024
_NPTS = 49
_NE = 224
_ROWS = 21760


_BB = 128


def _corner_vals(e, bx1, by1, bin_w, bin_h, wf, wi, offset):
    kk = e & 3
    pq = e >> 2
    owi = pq % 7
    ohi = pq // 7

    xs = bx1 + (owi.astype(jnp.float32) + 0.5) * bin_w
    ys = by1 + (ohi.astype(jnp.float32) + 0.5) * bin_h
    valid = ((ys >= -1.0) & (ys <= wf) & (xs >= -1.0) & (xs <= wf)
             & (pq < _NPTS))

    x = jnp.clip(xs, 0.0, wf - 1.0)
    y = jnp.clip(ys, 0.0, wf - 1.0)
    x0 = jnp.floor(x)
    y0 = jnp.floor(y)
    lx = x - x0
    ly = y - y0

    kx = kk & 1
    ky = kk >> 1
    xsel = jnp.where(kx == 1, jnp.minimum(x0 + 1.0, wf - 1.0), x0)
    ysel = jnp.where(ky == 1, jnp.minimum(y0 + 1.0, wf - 1.0), y0)
    wgt = (jnp.where(kx == 1, lx, 1.0 - lx)
           * jnp.where(ky == 1, ly, 1.0 - ly))
    wgt = jnp.where(valid, wgt, 0.0)

    row = offset + ysel.astype(jnp.int32) * wi + xsel.astype(jnp.int32)
    return row, wgt


def _prep_body(boxes_ref, idx_ref, w_ref):
    x1 = boxes_ref[:, 0:1]
    y1 = boxes_ref[:, 1:2]
    x2 = boxes_ref[:, 2:3]
    y2 = boxes_ref[:, 3:4]

    area = jnp.maximum((x2 - x1) * (y2 - y1), 1e-6)
    lvl = jnp.floor(4.0 + jnp.log2(jnp.sqrt(area) / 224.0 + 1e-8))
    lvl = jnp.clip(lvl, 2.0, 5.0)
    li = (lvl - 2.0).astype(jnp.int32)

    scale = jnp.where(li == 0, 0.25,
            jnp.where(li == 1, 0.125,
            jnp.where(li == 2, 0.0625, 0.03125)))
    wf = jnp.where(li == 0, 128.0,
         jnp.where(li == 1, 64.0,
         jnp.where(li == 2, 32.0, 16.0)))
    wi = jnp.where(li == 0, 128,
         jnp.where(li == 1, 64,
         jnp.where(li == 2, 32, 16)))
    offset = jnp.where(li == 0, 0,
             jnp.where(li == 1, 16384,
             jnp.where(li == 2, 20480, 21504)))

    bx1 = x1 * scale
    by1 = y1 * scale
    roi_w = jnp.maximum(x2 * scale - bx1, 1.0)
    roi_h = jnp.maximum(y2 * scale - by1, 1.0)
    bin_w = roi_w / 7.0
    bin_h = roi_h / 7.0

    e = lax.broadcasted_iota(jnp.int32, (_BB, _NE), 1)
    row, _ = _corner_vals(e, bx1, by1, bin_w, bin_h, wf, wi, offset)
    idx_ref[:] = row

    f = lax.broadcasted_iota(jnp.int32, (_BB, _NE * 16), 1)
    e2 = ((f >> 7) << 3) + ((f & 127) >> 4)
    _, wgt = _corner_vals(e2, bx1, by1, bin_w, bin_h, wf, wi, offset)
    w_ref[:] = wgt


def _prep(boxes_p):
    return pl.pallas_call(
        _prep_body,
        grid=(_NBP // _BB,),
        in_specs=[pl.BlockSpec((_BB, 4), lambda i: (i, 0))],
        out_specs=(
            pl.BlockSpec((_BB, _NE), lambda i: (i, 0)),
            pl.BlockSpec((_BB, _NE * 16), lambda i: (i, 0)),
        ),
        out_shape=(
            jax.ShapeDtypeStruct((_NBP, _NE), jnp.int32),
            jax.ShapeDtypeStruct((_NBP, _NE * 16), jnp.float32),
        ),
    )(boxes_p)


def _sc_body(table, idxs, ws, out, idx8, ws8, rows0, rows1, rows2, rows3,
             out0, out1, sg0, sg1, sg2, sg3, so0, so1):
    wid = lax.axis_index("s") * 2 + lax.axis_index("c")
    base_box = wid * 32
    rows_bks = (rows0, rows1, rows2, rows3)
    sg_bks = (sg0, sg1, sg2, sg3)
    out_bks = (out0, out1)
    so_bks = (so0, so1)

    def start_q(bb, qt, bk):
        pltpu.async_copy(table.at[idx8.at[bb, qt]], rows_bks[bk], sg_bks[bk])

    def drain_q(qt):
        pltpu.make_async_copy(table.at[pl.ds(0, 56)], rows_bks[qt],
                              sg_bks[qt]).wait()

    def wait_out(ob):
        pltpu.make_async_copy(out_bks[ob], out.at[0], so_bks[ob]).wait()

    def compute_q(bb, qt, ob):
        rows_bk = rows_bks[qt]
        out_bk = out_bks[ob]

        def pt_body(p, carry2):
            base = p * 4
            pb = qt * 14 + p
            q = pb >> 1
            woff = (pb & 1) * 64
            w0 = ws8[bb, q, pl.ds(woff, 16)]
            w1 = ws8[bb, q, pl.ds(woff + 16, 16)]
            w2 = ws8[bb, q, pl.ds(woff + 32, 16)]
            w3 = ws8[bb, q, pl.ds(woff + 48, 16)]
            for j in range(16):
                sl = pl.ds(j * 16, 16)
                acc = w0 * rows_bk[base, sl]
                acc = acc + w1 * rows_bk[base + 1, sl]
                acc = acc + w2 * rows_bk[base + 2, sl]
                acc = acc + w3 * rows_bk[base + 3, sl]
                out_bk[p, sl] = acc
            return carry2

        lax.fori_loop(0, 14, pt_body, 0)

    def group_body(g, carry):
        n0 = base_box + 8 * g
        pltpu.sync_copy(idxs.at[pl.ds(n0, 8)], idx8)
        pltpu.sync_copy(ws.at[pl.ds(n0, 8)], ws8)
        start_q(0, 0, 0)
        start_q(0, 1, 1)
        start_q(0, 2, 2)

        def box_body(b, carry2):
            for qt in range(4):
                drain_q(qt)
                if qt == 0:
                    start_q(b, 3, 3)
                else:

                    @pl.when(b < 7)
                    def _():
                        start_q(b + 1, qt - 1, qt - 1)

                ob = qt & 1

                @pl.when(32 * g + 4 * b + qt >= 2)
                def _():
                    wait_out(ob)

                compute_q(b, qt, ob)
                pltpu.async_copy(out_bks[ob],
                                 out.at[(n0 + b) * 4 + qt], so_bks[ob])
            return carry2

        lax.fori_loop(0, 8, box_body, 0)
        return carry

    lax.fori_loop(0, 4, group_body, 0)
    wait_out(0)
    wait_out(1)


@functools.cache
def _sc_gather():
    return pl.kernel(
        _sc_body,
        mesh=plsc.VectorSubcoreMesh(core_axis_name="c", subcore_axis_name="s"),
        out_type=jax.ShapeDtypeStruct((_NBP * 4, 14, _C), jnp.float32),
        scratch_types=[
            pltpu.VMEM((8, 4, 56), jnp.int32),
            pltpu.VMEM((8, 28, 128), jnp.float32),
            pltpu.VMEM((56, _C), jnp.float32),
            pltpu.VMEM((56, _C), jnp.float32),
            pltpu.VMEM((56, _C), jnp.float32),
            pltpu.VMEM((56, _C), jnp.float32),
            pltpu.VMEM((14, _C), jnp.float32),
            pltpu.VMEM((14, _C), jnp.float32),
            pltpu.SemaphoreType.DMA,
            pltpu.SemaphoreType.DMA,
            pltpu.SemaphoreType.DMA,
            pltpu.SemaphoreType.DMA,
            pltpu.SemaphoreType.DMA,
            pltpu.SemaphoreType.DMA,
        ],
    )


def kernel(feat0, feat1, feat2, feat3, boxes):
    feats = [feat0[0], feat1[0], feat2[0], feat3[0]]
    table = jnp.concatenate(
        [jnp.transpose(f.reshape(_C, -1)) for f in feats], axis=0)
    boxes_p = jnp.zeros((_NBP, 4), jnp.float32).at[:_NB].set(boxes)
    idx, w = _prep(boxes_p)
    out = _sc_gather()(table, idx.reshape(_NBP, 4, 56),
                       w.reshape(_NBP, 28, 128))
    out = out.reshape(_NBP, 56, _C)[:_NB, :_NPTS]
    return jnp.transpose(out, (0, 2, 1)).reshape(_NB, _C, 7, 7)


# speedup vs baseline: 1.1067x; 1.1067x over previous
import functools

import jax
import jax.numpy as jnp
import numpy as np
from jax import lax
from jax.experimental import pallas as pl
from jax.experimental.pallas import tpu as pltpu
from jax.experimental.pallas import tpu_sc as plsc

_C = 256
_NB = 1000
_NBP = 1024
_NPTS = 49
_NE = 208
_ROWS = 21760


_BB = 128


def _corner_vals(e, bx1, by1, bin_w, bin_h, wf, wi, offset):
    kk = e & 3
    pq = e >> 2
    owi = pq % 7
    ohi = pq // 7

    xs = bx1 + (owi.astype(jnp.float32) + 0.5) * bin_w
    ys = by1 + (ohi.astype(jnp.float32) + 0.5) * bin_h
    valid = ((ys >= -1.0) & (ys <= wf) & (xs >= -1.0) & (xs <= wf)
             & (pq < _NPTS))

    x = jnp.clip(xs, 0.0, wf - 1.0)
    y = jnp.clip(ys, 0.0, wf - 1.0)
    x0 = jnp.floor(x)
    y0 = jnp.floor(y)
    lx = x - x0
    ly = y - y0

    kx = kk & 1
    ky = kk >> 1
    xsel = jnp.where(kx == 1, jnp.minimum(x0 + 1.0, wf - 1.0), x0)
    ysel = jnp.where(ky == 1, jnp.minimum(y0 + 1.0, wf - 1.0), y0)
    wgt = (jnp.where(kx == 1, lx, 1.0 - lx)
           * jnp.where(ky == 1, ly, 1.0 - ly))
    wgt = jnp.where(valid, wgt, 0.0)

    row = offset + ysel.astype(jnp.int32) * wi + xsel.astype(jnp.int32)
    return row, wgt


def _prep_body(boxes_ref, idx_ref, w_ref):
    x1 = boxes_ref[:, 0:1]
    y1 = boxes_ref[:, 1:2]
    x2 = boxes_ref[:, 2:3]
    y2 = boxes_ref[:, 3:4]

    area = jnp.maximum((x2 - x1) * (y2 - y1), 1e-6)
    lvl = jnp.floor(4.0 + jnp.log2(jnp.sqrt(area) / 224.0 + 1e-8))
    lvl = jnp.clip(lvl, 2.0, 5.0)
    li = (lvl - 2.0).astype(jnp.int32)

    scale = jnp.where(li == 0, 0.25,
            jnp.where(li == 1, 0.125,
            jnp.where(li == 2, 0.0625, 0.03125)))
    wf = jnp.where(li == 0, 128.0,
         jnp.where(li == 1, 64.0,
         jnp.where(li == 2, 32.0, 16.0)))
    wi = jnp.where(li == 0, 128,
         jnp.where(li == 1, 64,
         jnp.where(li == 2, 32, 16)))
    offset = jnp.where(li == 0, 0,
             jnp.where(li == 1, 16384,
             jnp.where(li == 2, 20480, 21504)))

    bx1 = x1 * scale
    by1 = y1 * scale
    roi_w = jnp.maximum(x2 * scale - bx1, 1.0)
    roi_h = jnp.maximum(y2 * scale - by1, 1.0)
    bin_w = roi_w / 7.0
    bin_h = roi_h / 7.0

    e = lax.broadcasted_iota(jnp.int32, (_BB, _NE), 1)
    row, _ = _corner_vals(e, bx1, by1, bin_w, bin_h, wf, wi, offset)
    idx_ref[:] = row

    f = lax.broadcasted_iota(jnp.int32, (_BB, _NE * 16), 1)
    e2 = ((f >> 7) << 3) + ((f & 127) >> 4)
    _, wgt = _corner_vals(e2, bx1, by1, bin_w, bin_h, wf, wi, offset)
    w_ref[:] = wgt


def _prep(boxes_p):
    return pl.pallas_call(
        _prep_body,
        grid=(_NBP // _BB,),
        in_specs=[pl.BlockSpec((_BB, 4), lambda i: (i, 0))],
        out_specs=(
            pl.BlockSpec((_BB, _NE), lambda i: (i, 0)),
            pl.BlockSpec((_BB, _NE * 16), lambda i: (i, 0)),
        ),
        out_shape=(
            jax.ShapeDtypeStruct((_NBP, _NE), jnp.int32),
            jax.ShapeDtypeStruct((_NBP, _NE * 16), jnp.float32),
        ),
    )(boxes_p)


_NBH = 512


def _sc_body(table, idxs, ws, out, idx4, ws4, rows0, rows1, rows2,
             out0, out1, sg0, sg1, sg2, so0, so1):
    wid = lax.axis_index("s") * 2 + lax.axis_index("c")
    base_box = wid * (_NBH // 32)
    rows_bks = (rows0, rows1, rows2)
    sg_bks = (sg0, sg1, sg2)
    out_bks = (out0, out1)
    so_bks = (so0, so1)

    def start_half(bb, half, hh):
        pltpu.async_copy(table.at[idx4.at[bb, half]], rows_bks[hh % 3],
                         sg_bks[hh % 3])

    def drain_gather(hh):
        pltpu.make_async_copy(table.at[pl.ds(0, 104)], rows_bks[hh % 3],
                              sg_bks[hh % 3]).wait()

    def wait_out(ob):
        pltpu.make_async_copy(out_bks[ob], out.at[0], so_bks[ob]).wait()

    def compute_half(bb, half, hh, ob):
        rows_bk = rows_bks[hh % 3]
        out_bk = out_bks[ob]

        def pt_body(p, carry2):
            base = p * 4
            q = half * 13 + (p >> 1)
            woff = (p & 1) * 64
            w0 = ws4[bb, q, pl.ds(woff, 16)]
            w1 = ws4[bb, q, pl.ds(woff + 16, 16)]
            w2 = ws4[bb, q, pl.ds(woff + 32, 16)]
            w3 = ws4[bb, q, pl.ds(woff + 48, 16)]
            for j in range(16):
                sl = pl.ds(j * 16, 16)
                acc = w0 * rows_bk[base, sl]
                acc = acc + w1 * rows_bk[base + 1, sl]
                acc = acc + w2 * rows_bk[base + 2, sl]
                acc = acc + w3 * rows_bk[base + 3, sl]
                out_bk[p + half * 26, sl] = acc
            return carry2

        lax.fori_loop(0, 26, pt_body, 0)

    def group_body(g, carry):
        n0 = base_box + 4 * g
        pltpu.sync_copy(idxs.at[pl.ds(n0, 4)], idx4)
        pltpu.sync_copy(ws.at[pl.ds(n0, 4)], ws4)
        start_half(0, 0, 0)
        start_half(0, 1, 1)
        for hh in range(8):
            bb = hh >> 1
            half = hh & 1
            ob = bb & 1
            drain_gather(hh)
            if hh < 6:
                start_half((hh + 2) >> 1, (hh + 2) & 1, hh + 2)
            if half == 0:
                @pl.when(4 * g + bb >= 2)
                def _():
                    wait_out(ob)
            compute_half(bb, half, hh, ob)
            if half == 1:
                pltpu.async_copy(out_bks[ob], out.at[n0 + bb], so_bks[ob])
        return carry

    lax.fori_loop(0, _NBH // 128, group_body, 0)
    wait_out(0)
    wait_out(1)


@functools.cache
def _sc_gather():
    return pl.kernel(
        _sc_body,
        mesh=plsc.VectorSubcoreMesh(core_axis_name="c", subcore_axis_name="s"),
        out_type=jax.ShapeDtypeStruct((_NBH, 52, _C), jnp.float32),
        scratch_types=[
            pltpu.VMEM((4, 2, 104), jnp.int32),
            pltpu.VMEM((4, 26, 128), jnp.float32),
            pltpu.VMEM((104, _C), jnp.float32),
            pltpu.VMEM((104, _C), jnp.float32),
            pltpu.VMEM((104, _C), jnp.float32),
            pltpu.VMEM((52, _C), jnp.float32),
            pltpu.VMEM((52, _C), jnp.float32),
            pltpu.SemaphoreType.DMA,
            pltpu.SemaphoreType.DMA,
            pltpu.SemaphoreType.DMA,
            pltpu.SemaphoreType.DMA,
            pltpu.SemaphoreType.DMA,
        ],
    )


def _tr_body(in_ref, eye_ref, out_ref):
    for b in range(8):
        out_ref[b] = jax.lax.dot_general(
            in_ref[b], eye_ref[...], (((0,), (0,)), ((), ())),
            precision=jax.lax.Precision.HIGHEST)


def _tr_body2(in_ref, eye_ref, alias_ref, out_ref):
    del alias_ref
    _tr_body(in_ref, eye_ref, out_ref)


def _transpose_chunk1(o1, eye):
    return pl.pallas_call(
        _tr_body,
        grid=(_NBH // 8,),
        in_specs=[
            pl.BlockSpec((8, 52, _C), lambda i: (i, 0, 0)),
            pl.BlockSpec((52, _NPTS), lambda i: (0, 0)),
        ],
        out_specs=pl.BlockSpec((8, _C, _NPTS), lambda i: (i, 0, 0)),
        out_shape=jax.ShapeDtypeStruct((_NB, _C, _NPTS), jnp.float32),
    )(o1, eye)


def _transpose_chunk2(o2, eye, t1):
    nblk = (_NB - _NBH) // 8
    return pl.pallas_call(
        _tr_body2,
        grid=(nblk,),
        in_specs=[
            pl.BlockSpec((8, 52, _C), lambda i: (i, 0, 0)),
            pl.BlockSpec((52, _NPTS), lambda i: (0, 0)),
            pl.BlockSpec(memory_space=pl.ANY),
        ],
        out_specs=pl.BlockSpec((8, _C, _NPTS),
                               lambda i: (i + _NBH // 8, 0, 0)),
        out_shape=jax.ShapeDtypeStruct((_NB, _C, _NPTS), jnp.float32),
        input_output_aliases={2: 0},
    )(o2, eye, t1)


def kernel(feat0, feat1, feat2, feat3, boxes):
    feats = [feat0[0], feat1[0], feat2[0], feat3[0]]
    table = jnp.concatenate(
        [jnp.transpose(f.reshape(_C, -1)) for f in feats], axis=0)
    boxes_p = jnp.zeros((_NBP, 4), jnp.float32).at[:_NB].set(boxes)
    idx, w = _prep(boxes_p)
    idx3 = idx.reshape(_NBP, 2, 104)
    wp = w.reshape(_NBP, 26, 128)
    sc = _sc_gather()
    o1 = sc(table, idx3[:_NBH], wp[:_NBH])
    o2 = sc(table, idx3[_NBH:], wp[_NBH:])
    eye = jnp.eye(52, _NPTS, dtype=jnp.float32)
    t1 = _transpose_chunk1(o1, eye)
    t_all = _transpose_chunk2(o2, eye, t1)
    return t_all.reshape(_NB, _C, 7, 7)
